# baseline (speedup 1.0000x reference)
import functools

import jax
import jax.numpy as jnp
from jax import lax
from jax.experimental import pallas as pl
from jax.experimental.pallas import tpu as pltpu
from jax.experimental.pallas import tpu_sc as plsc

NC = 2
NS = 16
NW = NC * NS
CH = 128
ZR = 128


def _seg_sum_call(n_pad, d, iters, nchunks):
    rpt = n_pad // NS
    assert n_pad % NS == 0 and rpt % ZR == 0 and d % 16 == 0
    assert iters % 2 == 0

    def body(*refs):
        src_hbm, dst_hbm = refs[0], refs[1]
        tabs = refs[2:2 + nchunks]
        outs = refs[2 + nchunks:2 + 2 * nchunks]
        (src_v, dst_v, rows0, rows1, zbuf, acc, sem0, sem1) = \
            refs[2 + 2 * nchunks:]
        c = lax.axis_index("c")
        s = lax.axis_index("s")
        w = s * NC + c
        row0 = s * rpt

        def zrow(r, _):
            def zcol(k, _):
                zbuf[r, pl.ds(k * 16, 16)] = jnp.zeros((16,), jnp.float32)
                return 0
            lax.fori_loop(0, d // 16, zcol, 0)
            return 0
        lax.fori_loop(0, ZR, zrow, 0)

        pltpu.sync_copy(src_hbm.at[pl.ds(w * iters, iters)], src_v)
        pltpu.sync_copy(dst_hbm.at[pl.ds(w * iters, iters)], dst_v)

        rows = (rows0, rows1)
        sems = (sem0, sem1)

        for tab_hbm, out_hbm in zip(tabs, outs):
            def zacc(k, _):
                pltpu.sync_copy(zbuf, acc.at[pl.ds(row0 + k * ZR, ZR)])
                return 0
            lax.fori_loop(0, rpt // ZR, zacc, 0)
            plsc.subcore_barrier()

            def wait_gather(k):
                pltpu.make_async_copy(tab_hbm.at[pl.ds(0, CH)], rows[k],
                                      sems[k]).wait()

            pltpu.async_copy(tab_hbm.at[src_v.at[0]], rows0, sem0)

            def edge(i, _):
                j = i * 2
                pltpu.async_copy(tab_hbm.at[src_v.at[j + 1]], rows1, sem1)
                wait_gather(0)
                pltpu.sync_copy(rows0, acc.at[dst_v.at[j]], add=True)

                @pl.when(i < iters // 2 - 1)
                def _():
                    pltpu.async_copy(tab_hbm.at[src_v.at[j + 2]], rows0,
                                     sem0)
                wait_gather(1)
                pltpu.sync_copy(rows1, acc.at[dst_v.at[j + 1]], add=True)
                return 0
            lax.fori_loop(0, iters // 2, edge, 0)

            plsc.subcore_barrier()
            pltpu.sync_copy(acc.at[pl.ds(row0, rpt)],
                            out_hbm.at[pl.ds(row0, rpt), pl.ds(c * d, d)])

    mesh = plsc.VectorSubcoreMesh(core_axis_name="c", subcore_axis_name="s",
                                  num_cores=NC, num_subcores=NS)
    return pl.kernel(
        body,
        out_type=[jax.ShapeDtypeStruct((n_pad, NC * d), jnp.float32)
                  for _ in range(nchunks)],
        mesh=mesh,
        scratch_types=[
            pltpu.VMEM((iters, CH), jnp.int32),
            pltpu.VMEM((iters, CH), jnp.int32),
            pltpu.VMEM((CH, d), jnp.float32),
            pltpu.VMEM((CH, d), jnp.float32),
            pltpu.VMEM((ZR, d), jnp.float32),
            pltpu.VMEM_SHARED((n_pad, d), jnp.float32),
            pltpu.SemaphoreType.DMA,
            pltpu.SemaphoreType.DMA,
        ],
        compiler_params=pltpu.CompilerParams(use_tc_tiling_on_sc=False),
    )


def _tc1a_body(x_ref, ws1_ref, b1_ref, hs_ref):
    hs_ref[...] = jnp.dot(x_ref[...], ws1_ref[...],
                          preferred_element_type=jnp.float32) + b1_ref[...]


def _make_tc1b_body(nchunks, d_chunk):
    def body(*refs):
        hs_ref = refs[0]
        p_refs = refs[1:1 + nchunks]
        (deg_ref, wn1_ref, ws2_ref, wn2_ref, b2_ref,
         t2_ref, s2_ref) = refs[1 + nchunks:]
        denom = jnp.maximum(deg_ref[...], 1.0)
        agg = jnp.concatenate(
            [p[:, :d_chunk] + p[:, d_chunk:] for p in p_refs],
            axis=-1) / denom
        h = hs_ref[...] + jnp.dot(agg, wn1_ref[...],
                                  preferred_element_type=jnp.float32)
        h = jnp.maximum(h, 0.0)
        t2_ref[...] = jnp.dot(h, wn2_ref[...],
                              preferred_element_type=jnp.float32)
        s2_ref[...] = jnp.dot(h, ws2_ref[...],
                              preferred_element_type=jnp.float32) + b2_ref[...]
    return body


def _make_tc2_body(d_out):
    def body(s2_ref, p_ref, deg_ref, o_ref):
        denom = jnp.maximum(deg_ref[...], 1.0)
        q = p_ref[:, :d_out] + p_ref[:, d_out:]
        o_ref[...] = s2_ref[...] + q / denom
    return body


def _pick_bm(n):
    for bm in (2000, 1024, 1000, 512, 400, 256, 200, 128, 80, 40, 16, 8):
        if n % bm == 0:
            return bm
    return n


def kernel(x, edge_index, in_degrees, W_self1, W_neigh1, b1, W_self2,
           W_neigh2, b2):
    n, d_in = x.shape
    d_hid = W_self1.shape[1]
    d_out = W_self2.shape[1]
    e = edge_index.shape[1]

    src = edge_index[0]
    dst = edge_index[1]
    epb = NW * CH * 8
    e_pad = ((e + epb - 1) // epb) * epb
    if e_pad != e:
        src = jnp.concatenate([src, jnp.zeros((e_pad - e,), jnp.int32)])
        dst = jnp.concatenate([dst, jnp.full((e_pad - e,), n, jnp.int32)])
    iters = e_pad // (NW * CH)
    n_pad = ((n + NS * ZR - 1) // (NS * ZR)) * (NS * ZR)
    src2 = src.reshape(NW * iters, CH)
    dst2 = dst.reshape(NW * iters, CH)
    degf = in_degrees.astype(jnp.float32).reshape(n, 1)

    bm = _pick_bm(n)
    grid = (n // bm,)

    hs = pl.pallas_call(
        _tc1a_body,
        grid=grid,
        in_specs=[
            pl.BlockSpec((bm, d_in), lambda i: (i, 0)),
            pl.BlockSpec((d_in, d_hid), lambda i: (0, 0)),
            pl.BlockSpec((1, d_hid), lambda i: (0, 0)),
        ],
        out_specs=pl.BlockSpec((bm, d_hid), lambda i: (i, 0)),
        out_shape=jax.ShapeDtypeStruct((n, d_hid), jnp.float32),
    )(x, W_self1, b1.reshape(1, d_hid))

    spmem_budget_words = 1_200_000
    max_d = max(16, (spmem_budget_words // n_pad) // 16 * 16)
    nchunks = -(-d_in // max_d)
    d_chunk = -(-(d_in // nchunks) // 16) * 16
    assert d_chunk * nchunks >= d_in and d_in % d_chunk == 0
    nchunks = d_in // d_chunk

    seg1 = _seg_sum_call(n_pad, d_chunk, iters, nchunks)
    xs = [lax.slice_in_dim(x, k * d_chunk, (k + 1) * d_chunk, axis=1)
          for k in range(nchunks)]
    p1s = seg1(src2, dst2, *xs)
    if not isinstance(p1s, (list, tuple)):
        p1s = [p1s]

    t2, s2 = pl.pallas_call(
        _make_tc1b_body(nchunks, d_chunk),
        grid=grid,
        in_specs=[
            pl.BlockSpec((bm, d_hid), lambda i: (i, 0)),
        ] + [
            pl.BlockSpec((bm, NC * d_chunk), lambda i: (i, 0))
            for _ in range(nchunks)
        ] + [
            pl.BlockSpec((bm, 1), lambda i: (i, 0)),
            pl.BlockSpec((d_in, d_hid), lambda i: (0, 0)),
            pl.BlockSpec((d_hid, d_out), lambda i: (0, 0)),
            pl.BlockSpec((d_hid, d_out), lambda i: (0, 0)),
            pl.BlockSpec((1, d_out), lambda i: (0, 0)),
        ],
        out_specs=[
            pl.BlockSpec((bm, d_out), lambda i: (i, 0)),
            pl.BlockSpec((bm, d_out), lambda i: (i, 0)),
        ],
        out_shape=[jax.ShapeDtypeStruct((n, d_out), jnp.float32)] * 2,
    )(hs, *p1s, degf, W_neigh1, W_self2, W_neigh2, b2.reshape(1, d_out))

    seg2 = _seg_sum_call(n_pad, d_out, iters, 1)
    p2 = seg2(src2, dst2, t2)
    if isinstance(p2, (list, tuple)):
        p2 = p2[0]

    out = pl.pallas_call(
        _make_tc2_body(d_out),
        grid=grid,
        in_specs=[
            pl.BlockSpec((bm, d_out), lambda i: (i, 0)),
            pl.BlockSpec((bm, NC * d_out), lambda i: (i, 0)),
            pl.BlockSpec((bm, 1), lambda i: (i, 0)),
        ],
        out_specs=pl.BlockSpec((bm, d_out), lambda i: (i, 0)),
        out_shape=jax.ShapeDtypeStruct((n, d_out), jnp.float32),
    )(s2, p2, degf)
    return out


# speedup vs baseline: 2.6252x; 2.6252x over previous
import functools

import jax
import jax.numpy as jnp
from jax import lax
from jax.experimental import pallas as pl
from jax.experimental.pallas import tpu as pltpu
from jax.experimental.pallas import tpu_sc as plsc

NC = 2
NS = 16
NW = NC * NS
CH = 125
ZR = 128


def _seg_sum_call(n_pad, d, iters, nchunks):
    rpt = n_pad // NS
    assert n_pad % NS == 0 and rpt % ZR == 0 and d % 16 == 0
    assert iters % 2 == 0

    def body(*refs):
        src_hbm, dst_hbm = refs[0], refs[1]
        tabs = refs[2:2 + nchunks]
        outs = refs[2 + nchunks:2 + 2 * nchunks]
        (src_v, dst_v, rows0, rows1, zbuf, acc, sem0, sem1) = \
            refs[2 + 2 * nchunks:]
        c = lax.axis_index("c")
        s = lax.axis_index("s")
        w = s * NC + c
        row0 = s * rpt

        def zrow(r, _):
            def zcol(k, _):
                zbuf[r, pl.ds(k * 16, 16)] = jnp.zeros((16,), jnp.float32)
                return 0
            lax.fori_loop(0, d // 16, zcol, 0)
            return 0
        lax.fori_loop(0, ZR, zrow, 0)

        pltpu.sync_copy(src_hbm.at[pl.ds(w * iters, iters)], src_v)
        pltpu.sync_copy(dst_hbm.at[pl.ds(w * iters, iters)], dst_v)

        rows = (rows0, rows1)
        sems = (sem0, sem1)

        for tab_hbm, out_hbm in zip(tabs, outs):
            def zacc(k, _):
                pltpu.sync_copy(zbuf, acc.at[pl.ds(row0 + k * ZR, ZR)])
                return 0
            lax.fori_loop(0, rpt // ZR, zacc, 0)
            plsc.subcore_barrier()

            def wait_gather(k):
                pltpu.make_async_copy(tab_hbm.at[pl.ds(0, CH)], rows[k],
                                      sems[k]).wait()

            pltpu.async_copy(tab_hbm.at[src_v.at[0]], rows0, sem0)

            def edge(i, _):
                j = i * 2
                pltpu.async_copy(tab_hbm.at[src_v.at[j + 1]], rows1, sem1)
                wait_gather(0)
                pltpu.sync_copy(rows0, acc.at[dst_v.at[j]], add=True)

                @pl.when(i < iters // 2 - 1)
                def _():
                    pltpu.async_copy(tab_hbm.at[src_v.at[j + 2]], rows0,
                                     sem0)
                wait_gather(1)
                pltpu.sync_copy(rows1, acc.at[dst_v.at[j + 1]], add=True)
                return 0
            lax.fori_loop(0, iters // 2, edge, 0)

            plsc.subcore_barrier()
            pltpu.sync_copy(acc.at[pl.ds(row0, rpt)],
                            out_hbm.at[c, pl.ds(row0, rpt)])

    mesh = plsc.VectorSubcoreMesh(core_axis_name="c", subcore_axis_name="s",
                                  num_cores=NC, num_subcores=NS)
    return pl.kernel(
        body,
        out_type=[jax.ShapeDtypeStruct((NC, n_pad, d), jnp.float32)
                  for _ in range(nchunks)],
        mesh=mesh,
        scratch_types=[
            pltpu.VMEM((iters, CH), jnp.int32),
            pltpu.VMEM((iters, CH), jnp.int32),
            pltpu.VMEM((CH, d), jnp.float32),
            pltpu.VMEM((CH, d), jnp.float32),
            pltpu.VMEM((ZR, d), jnp.float32),
            pltpu.VMEM_SHARED((n_pad, d), jnp.float32),
            pltpu.SemaphoreType.DMA,
            pltpu.SemaphoreType.DMA,
        ],
        compiler_params=pltpu.CompilerParams(use_tc_tiling_on_sc=False),
    )


def _tc1a_body(x_ref, ws1_ref, b1_ref, hs_ref):
    hs_ref[...] = jnp.dot(x_ref[...], ws1_ref[...],
                          preferred_element_type=jnp.float32) + b1_ref[...]


def _make_tc1b_body(nchunks, d_chunk):
    def body(*refs):
        hs_ref = refs[0]
        p_refs = refs[1:1 + nchunks]
        (deg_ref, wn1_ref, ws2_ref, wn2_ref, b2_ref,
         t2_ref, s2_ref) = refs[1 + nchunks:]
        denom = jnp.maximum(deg_ref[...], 1.0)
        agg = jnp.concatenate([p[0] + p[1] for p in p_refs],
                              axis=-1) / denom
        h = hs_ref[...] + jnp.dot(agg, wn1_ref[...],
                                  preferred_element_type=jnp.float32)
        h = jnp.maximum(h, 0.0)
        t2_ref[...] = jnp.dot(h, wn2_ref[...],
                              preferred_element_type=jnp.float32)
        s2_ref[...] = jnp.dot(h, ws2_ref[...],
                              preferred_element_type=jnp.float32) + b2_ref[...]
    return body


def _make_tc2_body(d_out):
    def body(s2_ref, p_ref, deg_ref, o_ref):
        denom = jnp.maximum(deg_ref[...], 1.0)
        q = p_ref[0] + p_ref[1]
        o_ref[...] = s2_ref[...] + q / denom
    return body


def _pick_bm(n):
    for bm in (2000, 1024, 1000, 512, 400, 256, 200, 128, 80, 40, 16, 8):
        if n % bm == 0:
            return bm
    return n


def kernel(x, edge_index, in_degrees, W_self1, W_neigh1, b1, W_self2,
           W_neigh2, b2):
    n, d_in = x.shape
    d_hid = W_self1.shape[1]
    d_out = W_self2.shape[1]
    e = edge_index.shape[1]

    src = edge_index[0]
    dst = edge_index[1]
    epb = NW * CH * 8
    e_pad = ((e + epb - 1) // epb) * epb
    if e_pad != e:
        src = jnp.concatenate([src, jnp.zeros((e_pad - e,), jnp.int32)])
        dst = jnp.concatenate([dst, jnp.full((e_pad - e,), n, jnp.int32)])
    iters = e_pad // (NW * CH)
    n_pad = ((n + NS * ZR - 1) // (NS * ZR)) * (NS * ZR)
    src2 = src.reshape(NW * iters, CH)
    dst2 = dst.reshape(NW * iters, CH)
    degf = in_degrees.astype(jnp.float32).reshape(n, 1)

    bm = _pick_bm(n)
    grid = (n // bm,)

    hs = pl.pallas_call(
        _tc1a_body,
        grid=grid,
        in_specs=[
            pl.BlockSpec((bm, d_in), lambda i: (i, 0)),
            pl.BlockSpec((d_in, d_hid), lambda i: (0, 0)),
            pl.BlockSpec((1, d_hid), lambda i: (0, 0)),
        ],
        out_specs=pl.BlockSpec((bm, d_hid), lambda i: (i, 0)),
        out_shape=jax.ShapeDtypeStruct((n, d_hid), jnp.float32),
    )(x, W_self1, b1.reshape(1, d_hid))

    spmem_budget_words = 1_200_000
    max_d = max(16, (spmem_budget_words // n_pad) // 16 * 16)
    nchunks = -(-d_in // max_d)
    d_chunk = -(-(d_in // nchunks) // 16) * 16
    assert d_chunk * nchunks >= d_in and d_in % d_chunk == 0
    nchunks = d_in // d_chunk

    seg1 = _seg_sum_call(n_pad, d_chunk, iters, nchunks)
    xs = [lax.slice_in_dim(x, k * d_chunk, (k + 1) * d_chunk, axis=1)
          for k in range(nchunks)]
    p1s = seg1(src2, dst2, *xs)
    if not isinstance(p1s, (list, tuple)):
        p1s = [p1s]

    t2, s2 = pl.pallas_call(
        _make_tc1b_body(nchunks, d_chunk),
        grid=grid,
        in_specs=[
            pl.BlockSpec((bm, d_hid), lambda i: (i, 0)),
        ] + [
            pl.BlockSpec((NC, bm, d_chunk), lambda i: (0, i, 0))
            for _ in range(nchunks)
        ] + [
            pl.BlockSpec((bm, 1), lambda i: (i, 0)),
            pl.BlockSpec((d_in, d_hid), lambda i: (0, 0)),
            pl.BlockSpec((d_hid, d_out), lambda i: (0, 0)),
            pl.BlockSpec((d_hid, d_out), lambda i: (0, 0)),
            pl.BlockSpec((1, d_out), lambda i: (0, 0)),
        ],
        out_specs=[
            pl.BlockSpec((bm, d_out), lambda i: (i, 0)),
            pl.BlockSpec((bm, d_out), lambda i: (i, 0)),
        ],
        out_shape=[jax.ShapeDtypeStruct((n, d_out), jnp.float32)] * 2,
    )(hs, *p1s, degf, W_neigh1, W_self2, W_neigh2, b2.reshape(1, d_out))

    seg2 = _seg_sum_call(n_pad, d_out, iters, 1)
    p2 = seg2(src2, dst2, t2)
    if isinstance(p2, (list, tuple)):
        p2 = p2[0]

    out = pl.pallas_call(
        _make_tc2_body(d_out),
        grid=grid,
        in_specs=[
            pl.BlockSpec((bm, d_out), lambda i: (i, 0)),
            pl.BlockSpec((NC, bm, d_out), lambda i: (0, i, 0)),
            pl.BlockSpec((bm, 1), lambda i: (i, 0)),
        ],
        out_specs=pl.BlockSpec((bm, d_out), lambda i: (i, 0)),
        out_shape=jax.ShapeDtypeStruct((n, d_out), jnp.float32),
    )(s2, p2, degf)
    return out


# speedup vs baseline: 3.0584x; 1.1650x over previous
import functools

import jax
import jax.numpy as jnp
from jax import lax
from jax.experimental import pallas as pl
from jax.experimental.pallas import tpu as pltpu
from jax.experimental.pallas import tpu_sc as plsc

NC = 2
NS = 16
NW = NC * NS
CH = 125
ZR = 64


def _seg_sum_call(n_pad, d, iters, nchunks):
    rpt = n_pad // NS
    assert n_pad % NS == 0 and rpt % ZR == 0 and d % 16 == 0

    NB = 8
    LA = NB // 2
    assert iters % NB == 0

    def body(*refs):
        src_hbm, dst_hbm = refs[0], refs[1]
        tabs = refs[2:2 + nchunks]
        outs = refs[2 + nchunks:2 + 2 * nchunks]
        rest = refs[2 + 2 * nchunks:]
        src_v, dst_v = rest[0], rest[1]
        rows = rest[2:2 + NB]
        zbuf, acc = rest[2 + NB], rest[3 + NB]
        gsems = rest[4 + NB:4 + 2 * NB]
        ssems = rest[4 + 2 * NB:4 + 3 * NB]
        c = lax.axis_index("c")
        s = lax.axis_index("s")
        w = s * NC + c
        row0 = s * rpt

        def zrow(r, _):
            def zcol(k, _):
                zbuf[r, pl.ds(k * 16, 16)] = jnp.zeros((16,), jnp.float32)
                return 0
            lax.fori_loop(0, d // 16, zcol, 0)
            return 0
        lax.fori_loop(0, ZR, zrow, 0)

        pltpu.sync_copy(src_hbm.at[pl.ds(w * iters, iters)], src_v)
        pltpu.sync_copy(dst_hbm.at[pl.ds(w * iters, iters)], dst_v)

        def wait_dma(sem, k):
            pltpu.make_async_copy(tabs[0].at[pl.ds(0, CH)], rows[k],
                                  sem).wait()

        G = iters // NB
        for ph, (tab_hbm, out_hbm) in enumerate(zip(tabs, outs)):
            def gather(j, k):
                pltpu.async_copy(tab_hbm.at[src_v.at[j]], rows[k], gsems[k])

            def scatter(j, k):
                pltpu.async_copy(rows[k], acc.at[dst_v.at[j]], ssems[k],
                                 add=True)

            for k in range(LA):
                gather(k, k)

            def zacc(k, _):
                pltpu.sync_copy(zbuf, acc.at[pl.ds(row0 + k * ZR, ZR)])
                return 0
            lax.fori_loop(0, rpt // ZR, zacc, 0)
            plsc.subcore_barrier()

            def group(g, _):
                for k in range(NB):
                    j = g * NB + k
                    ka = (k + LA) % NB
                    if k < LA:
                        @pl.when(g > 0)
                        def _():
                            wait_dma(ssems[ka], ka)
                        gather(j + LA, ka)
                    else:
                        @pl.when(g < G - 1)
                        def _():
                            wait_dma(ssems[ka], ka)
                            gather(j + LA, ka)
                    wait_dma(gsems[k], k)
                    scatter(j, k)
                return 0
            lax.fori_loop(0, G, group, 0)

            for k in range(NB):
                wait_dma(ssems[k], k)
            plsc.subcore_barrier()
            pltpu.sync_copy(acc.at[pl.ds(row0, rpt)],
                            out_hbm.at[c, pl.ds(row0, rpt)])

    mesh = plsc.VectorSubcoreMesh(core_axis_name="c", subcore_axis_name="s",
                                  num_cores=NC, num_subcores=NS)
    return pl.kernel(
        body,
        out_type=[jax.ShapeDtypeStruct((NC, n_pad, d), jnp.float32)
                  for _ in range(nchunks)],
        mesh=mesh,
        scratch_types=(
            [pltpu.VMEM((iters, CH), jnp.int32)] * 2
            + [pltpu.VMEM((CH, d), jnp.float32)] * 8
            + [pltpu.VMEM((ZR, d), jnp.float32),
               pltpu.VMEM_SHARED((n_pad, d), jnp.float32)]
            + [pltpu.SemaphoreType.DMA] * 16
        ),
        compiler_params=pltpu.CompilerParams(use_tc_tiling_on_sc=False),
    )


def _tc1a_body(x_ref, ws1_ref, b1_ref, hs_ref):
    hs_ref[...] = jnp.dot(x_ref[...], ws1_ref[...],
                          preferred_element_type=jnp.float32) + b1_ref[...]


def _make_tc1b_body(nchunks, d_chunk):
    def body(*refs):
        hs_ref = refs[0]
        p_refs = refs[1:1 + nchunks]
        (deg_ref, wn1_ref, ws2_ref, wn2_ref, b2_ref,
         t2_ref, s2_ref) = refs[1 + nchunks:]
        denom = jnp.maximum(deg_ref[...], 1.0)
        agg = jnp.concatenate([p[0] + p[1] for p in p_refs],
                              axis=-1) / denom
        h = hs_ref[...] + jnp.dot(agg, wn1_ref[...],
                                  preferred_element_type=jnp.float32)
        h = jnp.maximum(h, 0.0)
        t2_ref[...] = jnp.dot(h, wn2_ref[...],
                              preferred_element_type=jnp.float32)
        s2_ref[...] = jnp.dot(h, ws2_ref[...],
                              preferred_element_type=jnp.float32) + b2_ref[...]
    return body


def _make_tc2_body(d_out):
    def body(s2_ref, p_ref, deg_ref, o_ref):
        denom = jnp.maximum(deg_ref[...], 1.0)
        q = p_ref[0] + p_ref[1]
        o_ref[...] = s2_ref[...] + q / denom
    return body


def _pick_bm(n):
    for bm in (2000, 1024, 1000, 512, 400, 256, 200, 128, 80, 40, 16, 8):
        if n % bm == 0:
            return bm
    return n


def kernel(x, edge_index, in_degrees, W_self1, W_neigh1, b1, W_self2,
           W_neigh2, b2):
    n, d_in = x.shape
    d_hid = W_self1.shape[1]
    d_out = W_self2.shape[1]
    e = edge_index.shape[1]

    src = edge_index[0]
    dst = edge_index[1]
    epb = NW * CH * 8
    e_pad = ((e + epb - 1) // epb) * epb
    if e_pad != e:
        src = jnp.concatenate([src, jnp.zeros((e_pad - e,), jnp.int32)])
        dst = jnp.concatenate([dst, jnp.full((e_pad - e,), n, jnp.int32)])
    iters = e_pad // (NW * CH)
    n_pad = ((n + NS * ZR - 1) // (NS * ZR)) * (NS * ZR)
    src2 = src.reshape(NW * iters, CH)
    dst2 = dst.reshape(NW * iters, CH)
    degf = in_degrees.astype(jnp.float32).reshape(n, 1)

    bm = _pick_bm(n)
    grid = (n // bm,)

    hs = pl.pallas_call(
        _tc1a_body,
        grid=grid,
        in_specs=[
            pl.BlockSpec((bm, d_in), lambda i: (i, 0)),
            pl.BlockSpec((d_in, d_hid), lambda i: (0, 0)),
            pl.BlockSpec((1, d_hid), lambda i: (0, 0)),
        ],
        out_specs=pl.BlockSpec((bm, d_hid), lambda i: (i, 0)),
        out_shape=jax.ShapeDtypeStruct((n, d_hid), jnp.float32),
    )(x, W_self1, b1.reshape(1, d_hid))

    spmem_budget_words = 1_200_000
    max_d = max(16, (spmem_budget_words // n_pad) // 16 * 16)
    nchunks = -(-d_in // max_d)
    d_chunk = -(-(d_in // nchunks) // 16) * 16
    assert d_chunk * nchunks >= d_in and d_in % d_chunk == 0
    nchunks = d_in // d_chunk

    seg1 = _seg_sum_call(n_pad, d_chunk, iters, nchunks)
    xs = [lax.slice_in_dim(x, k * d_chunk, (k + 1) * d_chunk, axis=1)
          for k in range(nchunks)]
    p1s = seg1(src2, dst2, *xs)
    if not isinstance(p1s, (list, tuple)):
        p1s = [p1s]

    t2, s2 = pl.pallas_call(
        _make_tc1b_body(nchunks, d_chunk),
        grid=grid,
        in_specs=[
            pl.BlockSpec((bm, d_hid), lambda i: (i, 0)),
        ] + [
            pl.BlockSpec((NC, bm, d_chunk), lambda i: (0, i, 0))
            for _ in range(nchunks)
        ] + [
            pl.BlockSpec((bm, 1), lambda i: (i, 0)),
            pl.BlockSpec((d_in, d_hid), lambda i: (0, 0)),
            pl.BlockSpec((d_hid, d_out), lambda i: (0, 0)),
            pl.BlockSpec((d_hid, d_out), lambda i: (0, 0)),
            pl.BlockSpec((1, d_out), lambda i: (0, 0)),
        ],
        out_specs=[
            pl.BlockSpec((bm, d_out), lambda i: (i, 0)),
            pl.BlockSpec((bm, d_out), lambda i: (i, 0)),
        ],
        out_shape=[jax.ShapeDtypeStruct((n, d_out), jnp.float32)] * 2,
    )(hs, *p1s, degf, W_neigh1, W_self2, W_neigh2, b2.reshape(1, d_out))

    seg2 = _seg_sum_call(n_pad, d_out, iters, 1)
    p2 = seg2(src2, dst2, t2)
    if isinstance(p2, (list, tuple)):
        p2 = p2[0]

    out = pl.pallas_call(
        _make_tc2_body(d_out),
        grid=grid,
        in_specs=[
            pl.BlockSpec((bm, d_out), lambda i: (i, 0)),
            pl.BlockSpec((NC, bm, d_out), lambda i: (0, i, 0)),
            pl.BlockSpec((bm, 1), lambda i: (i, 0)),
        ],
        out_specs=pl.BlockSpec((bm, d_out), lambda i: (i, 0)),
        out_shape=jax.ShapeDtypeStruct((n, d_out), jnp.float32),
    )(s2, p2, degf)
    return out


# speedup vs baseline: 3.4887x; 1.1407x over previous
import functools

import jax
import jax.numpy as jnp
from jax import lax
from jax.experimental import pallas as pl
from jax.experimental.pallas import tpu as pltpu
from jax.experimental.pallas import tpu_sc as plsc

NC = 2
NS = 16
NW = NC * NS
CH = 125
ZR = 64


def _seg_sum_call(n_pad, d, iters, nchunks):
    rpt = n_pad // NS
    assert n_pad % NS == 0 and rpt % ZR == 0 and d % 16 == 0

    NB = 8
    LA = NB // 2
    assert iters % NB == 0

    def body(*refs):
        src_hbm, dst_hbm = refs[0], refs[1]
        tabs = refs[2:2 + nchunks]
        outs = refs[2 + nchunks:2 + 2 * nchunks]
        rest = refs[2 + 2 * nchunks:]
        src_v, dst_v = rest[0], rest[1]
        rows = rest[2:2 + NB]
        zbuf, acc = rest[2 + NB], rest[3 + NB]
        gsems = rest[4 + NB:4 + 2 * NB]
        ssems = rest[4 + 2 * NB:4 + 3 * NB]
        c = lax.axis_index("c")
        s = lax.axis_index("s")
        w = s * NC + c
        row0 = s * rpt

        def zrow(r, _):
            def zcol(k, _):
                zbuf[r, pl.ds(k * 16, 16)] = jnp.zeros((16,), jnp.float32)
                return 0
            lax.fori_loop(0, d // 16, zcol, 0)
            return 0
        lax.fori_loop(0, ZR, zrow, 0)

        pltpu.sync_copy(src_hbm.at[pl.ds(w * iters, iters)], src_v)
        pltpu.sync_copy(dst_hbm.at[pl.ds(w * iters, iters)], dst_v)

        def wait_dma(sem, k):
            pltpu.make_async_copy(tabs[0].at[pl.ds(0, CH)], rows[k],
                                  sem).wait()

        G = iters // NB
        for ph, (tab_hbm, out_hbm) in enumerate(zip(tabs, outs)):
            def gather(j, k):
                pltpu.async_copy(tab_hbm.at[src_v.at[j]], rows[k], gsems[k])

            def scatter(j, k):
                pltpu.async_copy(rows[k], acc.at[dst_v.at[j]], ssems[k],
                                 add=True)

            for k in range(LA):
                gather(k, k)

            def zacc(k, _):
                pltpu.sync_copy(zbuf, acc.at[pl.ds(row0 + k * ZR, ZR)])
                return 0
            lax.fori_loop(0, rpt // ZR, zacc, 0)
            plsc.subcore_barrier()

            def group(g, _):
                for k in range(NB):
                    j = g * NB + k
                    ka = (k + LA) % NB
                    if k < LA:
                        @pl.when(g > 0)
                        def _():
                            wait_dma(ssems[ka], ka)
                        gather(j + LA, ka)
                    else:
                        @pl.when(g < G - 1)
                        def _():
                            wait_dma(ssems[ka], ka)
                            gather(j + LA, ka)
                    wait_dma(gsems[k], k)
                    scatter(j, k)
                return 0
            lax.fori_loop(0, G, group, 0)

            for k in range(NB):
                wait_dma(ssems[k], k)
            plsc.subcore_barrier()
            pltpu.sync_copy(acc.at[pl.ds(row0, rpt)],
                            out_hbm.at[c, pl.ds(row0, rpt)])

    mesh = plsc.VectorSubcoreMesh(core_axis_name="c", subcore_axis_name="s",
                                  num_cores=NC, num_subcores=NS)
    return pl.kernel(
        body,
        out_type=[jax.ShapeDtypeStruct((NC, n_pad, d), jnp.float32)
                  for _ in range(nchunks)],
        mesh=mesh,
        scratch_types=(
            [pltpu.VMEM((iters, CH), jnp.int32)] * 2
            + [pltpu.VMEM((CH, d), jnp.float32)] * 8
            + [pltpu.VMEM((ZR, d), jnp.float32),
               pltpu.VMEM_SHARED((n_pad, d), jnp.float32)]
            + [pltpu.SemaphoreType.DMA] * 16
        ),
        compiler_params=pltpu.CompilerParams(use_tc_tiling_on_sc=False),
    )


def _tc1a_body(xp_ref, ws1_ref, b1_ref, hs_ref):
    hs_ref[...] = jnp.dot(xp_ref[...], ws1_ref[...],
                          preferred_element_type=jnp.float32) + b1_ref[...]


def _make_tc1b_body(nchunks):
    def body(*refs):
        hs_ref = refs[0]
        p_refs = refs[1:1 + nchunks]
        (dinv_ref, wn1_ref, ws2_ref, wn2_ref, b2_ref,
         t2_ref, s2_ref) = refs[1 + nchunks:]
        agg = jnp.concatenate([p[0] + p[1] for p in p_refs],
                              axis=-1) * dinv_ref[...]
        h = hs_ref[...] + jnp.dot(agg, wn1_ref[...],
                                  preferred_element_type=jnp.float32)
        h = jnp.maximum(h, 0.0)
        t2_ref[...] = jnp.dot(h, wn2_ref[...],
                              preferred_element_type=jnp.float32)
        s2_ref[...] = jnp.dot(h, ws2_ref[...],
                              preferred_element_type=jnp.float32) + b2_ref[...]
    return body


def _tc2_body(s2_ref, p_ref, dinv_ref, o_ref):
    o_ref[...] = s2_ref[...] + (p_ref[0] + p_ref[1]) * dinv_ref[...]


def _pair(w):
    z = jnp.zeros_like(w)
    return jnp.concatenate([jnp.concatenate([w, z], 1),
                            jnp.concatenate([z, w], 1)], 0)


def _pick_bm(n):
    for bm in (2000, 1024, 1000, 512, 400, 256, 200, 128, 80, 40, 16, 8):
        if n % bm == 0:
            return bm
    return n


def kernel(x, edge_index, in_degrees, W_self1, W_neigh1, b1, W_self2,
           W_neigh2, b2):
    n, d_in = x.shape
    d_hid = W_self1.shape[1]
    d_out = W_self2.shape[1]
    e = edge_index.shape[1]

    src = edge_index[0]
    dst = edge_index[1]
    epb = NW * CH * 8
    e_pad = ((e + epb - 1) // epb) * epb
    if e_pad != e:
        src = jnp.concatenate([src, jnp.zeros((e_pad - e,), jnp.int32)])
        dst = jnp.concatenate([dst, jnp.full((e_pad - e,), n, jnp.int32)])
    iters = e_pad // (NW * CH)
    n_pad = ((n + NS * ZR - 1) // (NS * ZR)) * (NS * ZR)
    src2 = src.reshape(NW * iters, CH)
    dst2 = dst.reshape(NW * iters, CH)
    degf = in_degrees.astype(jnp.float32).reshape(n, 1)

    bm = _pick_bm(n)
    bmh = bm // 2
    grid = (n // bm,)

    xp = jnp.reshape(x, (n // 2, 2 * d_in))
    dinv = 1.0 / jnp.maximum(in_degrees.astype(jnp.float32), 1.0)

    hs = pl.pallas_call(
        _tc1a_body,
        grid=grid,
        in_specs=[
            pl.BlockSpec((bmh, 2 * d_in), lambda i: (i, 0)),
            pl.BlockSpec((2 * d_in, 2 * d_hid), lambda i: (0, 0)),
            pl.BlockSpec((1, 2 * d_hid), lambda i: (0, 0)),
        ],
        out_specs=pl.BlockSpec((bmh, 2 * d_hid), lambda i: (i, 0)),
        out_shape=jax.ShapeDtypeStruct((n // 2, 2 * d_hid), jnp.float32),
    )(xp, _pair(W_self1), jnp.concatenate([b1, b1]).reshape(1, 2 * d_hid))

    spmem_budget_words = 1_200_000
    max_d = max(16, (spmem_budget_words // n_pad) // 16 * 16)
    nchunks = -(-d_in // max_d)
    d_chunk = -(-(d_in // nchunks) // 16) * 16
    assert d_chunk * nchunks >= d_in and d_in % d_chunk == 0
    nchunks = d_in // d_chunk

    seg1 = _seg_sum_call(n_pad, d_chunk, iters, nchunks)
    xs = [lax.slice_in_dim(x, k * d_chunk, (k + 1) * d_chunk, axis=1)
          for k in range(nchunks)]
    p1s = seg1(src2, dst2, *xs)
    if not isinstance(p1s, (list, tuple)):
        p1s = [p1s]
    p1s = [jnp.reshape(p, (NC, n_pad // 2, 2 * d_chunk)) for p in p1s]

    wn1_rows = []
    zc = jnp.zeros((d_chunk, d_hid), jnp.float32)
    for k in range(nchunks):
        wk = W_neigh1[k * d_chunk:(k + 1) * d_chunk]
        wn1_rows.append(jnp.concatenate([wk, zc], 1))
        wn1_rows.append(jnp.concatenate([zc, wk], 1))
    wn1p = jnp.concatenate(wn1_rows, 0)
    dinv1 = jnp.tile(
        jnp.reshape(jnp.broadcast_to(dinv[:, None], (n, d_chunk)),
                    (n // 2, 2 * d_chunk)), (1, nchunks))

    t2p, s2p = pl.pallas_call(
        _make_tc1b_body(nchunks),
        grid=grid,
        in_specs=[
            pl.BlockSpec((bmh, 2 * d_hid), lambda i: (i, 0)),
        ] + [
            pl.BlockSpec((NC, bmh, 2 * d_chunk), lambda i: (0, i, 0))
            for _ in range(nchunks)
        ] + [
            pl.BlockSpec((bmh, 2 * d_in), lambda i: (i, 0)),
            pl.BlockSpec((2 * d_in, 2 * d_hid), lambda i: (0, 0)),
            pl.BlockSpec((2 * d_hid, 2 * d_out), lambda i: (0, 0)),
            pl.BlockSpec((2 * d_hid, 2 * d_out), lambda i: (0, 0)),
            pl.BlockSpec((1, 2 * d_out), lambda i: (0, 0)),
        ],
        out_specs=[
            pl.BlockSpec((bmh, 2 * d_out), lambda i: (i, 0)),
            pl.BlockSpec((bmh, 2 * d_out), lambda i: (i, 0)),
        ],
        out_shape=[jax.ShapeDtypeStruct((n // 2, 2 * d_out), jnp.float32)] * 2,
    )(hs, *p1s, dinv1, wn1p, _pair(W_self2), _pair(W_neigh2),
      jnp.concatenate([b2, b2]).reshape(1, 2 * d_out))

    t2 = jnp.reshape(t2p, (n, d_out))
    seg2 = _seg_sum_call(n_pad, d_out, iters, 1)
    p2 = seg2(src2, dst2, t2)
    if isinstance(p2, (list, tuple)):
        p2 = p2[0]
    p2 = jnp.reshape(p2, (NC, n_pad // 2, 2 * d_out))
    dinv2 = jnp.reshape(jnp.broadcast_to(dinv[:, None], (n, d_out)),
                        (n // 2, 2 * d_out))

    outp = pl.pallas_call(
        _tc2_body,
        grid=grid,
        in_specs=[
            pl.BlockSpec((bmh, 2 * d_out), lambda i: (i, 0)),
            pl.BlockSpec((NC, bmh, 2 * d_out), lambda i: (0, i, 0)),
            pl.BlockSpec((bmh, 2 * d_out), lambda i: (i, 0)),
        ],
        out_specs=pl.BlockSpec((bmh, 2 * d_out), lambda i: (i, 0)),
        out_shape=jax.ShapeDtypeStruct((n // 2, 2 * d_out), jnp.float32),
    )(s2p, p2, dinv2)
    return jnp.reshape(outp, (n, d_out))


# speedup vs baseline: 3.4902x; 1.0004x over previous
import jax
import jax.numpy as jnp
from jax import lax
from jax.experimental import pallas as pl
from jax.experimental.pallas import tpu as pltpu
from jax.experimental.pallas import tpu_sc as plsc

NC = 2
NS = 16
NW = NC * NS
CH = 125
ZR = 64


def _seg_sum_call(n_pad, d, iters, nchunks):
    rpt = n_pad // NS
    assert n_pad % NS == 0 and rpt % ZR == 0 and d % 16 == 0

    NB = 8
    LA = NB // 2
    assert iters % NB == 0

    def body(*refs):
        src_hbm, dst_hbm = refs[0], refs[1]
        tabs = refs[2:2 + nchunks]
        outs = refs[2 + nchunks:2 + 2 * nchunks]
        rest = refs[2 + 2 * nchunks:]
        src_v, dst_v = rest[0], rest[1]
        rows = rest[2:2 + NB]
        zbuf, acc = rest[2 + NB], rest[3 + NB]
        gsems = rest[4 + NB:4 + 2 * NB]
        ssems = rest[4 + 2 * NB:4 + 3 * NB]
        c = lax.axis_index("c")
        s = lax.axis_index("s")
        w = s * NC + c
        row0 = s * rpt

        def zrow(r, _):
            def zcol(k, _):
                zbuf[r, pl.ds(k * 16, 16)] = jnp.zeros((16,), jnp.float32)
                return 0
            lax.fori_loop(0, d // 16, zcol, 0)
            return 0
        lax.fori_loop(0, ZR, zrow, 0)

        pltpu.sync_copy(src_hbm.at[pl.ds(w * iters, iters)], src_v)
        pltpu.sync_copy(dst_hbm.at[pl.ds(w * iters, iters)], dst_v)

        def wait_dma(sem, k):
            pltpu.make_async_copy(tabs[0].at[pl.ds(0, CH)], rows[k],
                                  sem).wait()

        G = iters // NB
        for ph, (tab_hbm, out_hbm) in enumerate(zip(tabs, outs)):
            def gather(j, k):
                pltpu.async_copy(tab_hbm.at[src_v.at[j]], rows[k], gsems[k])

            def scatter(j, k):
                pltpu.async_copy(rows[k], acc.at[dst_v.at[j]], ssems[k],
                                 add=True)

            for k in range(LA):
                gather(k, k)

            def zacc(k, _):
                pltpu.sync_copy(zbuf, acc.at[pl.ds(row0 + k * ZR, ZR)])
                return 0
            lax.fori_loop(0, rpt // ZR, zacc, 0)
            plsc.subcore_barrier()

            def group(g, _):
                for k in range(NB):
                    j = g * NB + k
                    ka = (k + LA) % NB
                    if k < LA:
                        @pl.when(g > 0)
                        def _():
                            wait_dma(ssems[ka], ka)
                        gather(j + LA, ka)
                    else:
                        @pl.when(g < G - 1)
                        def _():
                            wait_dma(ssems[ka], ka)
                            gather(j + LA, ka)
                    wait_dma(gsems[k], k)
                    scatter(j, k)
                return 0
            lax.fori_loop(0, G, group, 0)

            for k in range(NB):
                wait_dma(ssems[k], k)
            plsc.subcore_barrier()
            pltpu.sync_copy(acc.at[pl.ds(row0, rpt)],
                            out_hbm.at[c, pl.ds(row0, rpt)])

    mesh = plsc.VectorSubcoreMesh(core_axis_name="c", subcore_axis_name="s",
                                  num_cores=NC, num_subcores=NS)
    return pl.kernel(
        body,
        out_type=[jax.ShapeDtypeStruct((NC, n_pad, d), jnp.float32)
                  for _ in range(nchunks)],
        mesh=mesh,
        scratch_types=(
            [pltpu.VMEM((iters, CH), jnp.int32)] * 2
            + [pltpu.VMEM((CH, d), jnp.float32)] * 8
            + [pltpu.VMEM((ZR, d), jnp.float32),
               pltpu.VMEM_SHARED((n_pad, d), jnp.float32)]
            + [pltpu.SemaphoreType.DMA] * 16
        ),
        compiler_params=pltpu.CompilerParams(use_tc_tiling_on_sc=False),
    )


def _tc1a_body(xp_ref, ws1_ref, b1_ref, hs_ref):
    hs_ref[...] = jnp.dot(xp_ref[...], ws1_ref[...],
                          preferred_element_type=jnp.float32) + b1_ref[...]


def _make_tc1b_body(nchunks):
    def body(*refs):
        hs_ref = refs[0]
        p_refs = refs[1:1 + nchunks]
        (dinv_ref, wn1_ref, ws2_ref, wn2_ref, b2_ref,
         t2_ref, s2_ref) = refs[1 + nchunks:]
        agg = jnp.concatenate([p[0] + p[1] for p in p_refs],
                              axis=-1) * dinv_ref[...]
        h = hs_ref[...] + jnp.dot(agg, wn1_ref[...],
                                  preferred_element_type=jnp.float32)
        h = jnp.maximum(h, 0.0)
        t2_ref[...] = jnp.dot(h, wn2_ref[...],
                              preferred_element_type=jnp.float32)
        s2_ref[...] = jnp.dot(h, ws2_ref[...],
                              preferred_element_type=jnp.float32) + b2_ref[...]
    return body


def _tc2_body(s2_ref, p_ref, dinv_ref, o_ref):
    o_ref[...] = s2_ref[...] + (p_ref[0] + p_ref[1]) * dinv_ref[...]


def _pair(w):
    z = jnp.zeros_like(w)
    return jnp.concatenate([jnp.concatenate([w, z], 1),
                            jnp.concatenate([z, w], 1)], 0)


def _pick_bm(n):
    for bm in (2000, 1024, 1000, 512, 400, 256, 200, 128, 80, 40, 16, 8):
        if n % bm == 0:
            return bm
    return n


def kernel(x, edge_index, in_degrees, W_self1, W_neigh1, b1, W_self2,
           W_neigh2, b2):
    n, d_in = x.shape
    d_hid = W_self1.shape[1]
    d_out = W_self2.shape[1]
    e = edge_index.shape[1]

    src = edge_index[0]
    dst = edge_index[1]
    epb = NW * CH * 8
    e_pad = ((e + epb - 1) // epb) * epb
    if e_pad != e:
        src = jnp.concatenate([src, jnp.zeros((e_pad - e,), jnp.int32)])
        dst = jnp.concatenate([dst, jnp.full((e_pad - e,), n, jnp.int32)])
    iters = e_pad // (NW * CH)
    n_pad = ((n + NS * ZR - 1) // (NS * ZR)) * (NS * ZR)
    src2 = src.reshape(NW * iters, CH)
    dst2 = dst.reshape(NW * iters, CH)

    bm = _pick_bm(n)
    bmh = bm // 2
    grid = (n // bm,)

    xp = jnp.reshape(x, (n // 2, 2 * d_in))
    dinv = 1.0 / jnp.maximum(in_degrees.astype(jnp.float32), 1.0)

    hs = pl.pallas_call(
        _tc1a_body,
        grid=grid,
        in_specs=[
            pl.BlockSpec((bmh, 2 * d_in), lambda i: (i, 0)),
            pl.BlockSpec((2 * d_in, 2 * d_hid), lambda i: (0, 0)),
            pl.BlockSpec((1, 2 * d_hid), lambda i: (0, 0)),
        ],
        out_specs=pl.BlockSpec((bmh, 2 * d_hid), lambda i: (i, 0)),
        out_shape=jax.ShapeDtypeStruct((n // 2, 2 * d_hid), jnp.float32),
    )(xp, _pair(W_self1), jnp.concatenate([b1, b1]).reshape(1, 2 * d_hid))

    spmem_budget_words = 1_200_000
    max_d = max(16, (spmem_budget_words // n_pad) // 16 * 16)
    nchunks = -(-d_in // max_d)
    d_chunk = -(-(d_in // nchunks) // 16) * 16
    assert d_chunk * nchunks >= d_in and d_in % d_chunk == 0
    nchunks = d_in // d_chunk

    seg1 = _seg_sum_call(n_pad, d_chunk, iters, nchunks)
    xs = [lax.slice_in_dim(x, k * d_chunk, (k + 1) * d_chunk, axis=1)
          for k in range(nchunks)]
    p1s = seg1(src2, dst2, *xs)
    if not isinstance(p1s, (list, tuple)):
        p1s = [p1s]
    p1s = [jnp.reshape(p, (NC, n_pad // 2, 2 * d_chunk)) for p in p1s]

    wn1_rows = []
    zc = jnp.zeros((d_chunk, d_hid), jnp.float32)
    for k in range(nchunks):
        wk = W_neigh1[k * d_chunk:(k + 1) * d_chunk]
        wn1_rows.append(jnp.concatenate([wk, zc], 1))
        wn1_rows.append(jnp.concatenate([zc, wk], 1))
    wn1p = jnp.concatenate(wn1_rows, 0)
    dinv1 = jnp.tile(
        jnp.reshape(jnp.broadcast_to(dinv[:, None], (n, d_chunk)),
                    (n // 2, 2 * d_chunk)), (1, nchunks))

    t2p, s2p = pl.pallas_call(
        _make_tc1b_body(nchunks),
        grid=grid,
        in_specs=[
            pl.BlockSpec((bmh, 2 * d_hid), lambda i: (i, 0)),
        ] + [
            pl.BlockSpec((NC, bmh, 2 * d_chunk), lambda i: (0, i, 0))
            for _ in range(nchunks)
        ] + [
            pl.BlockSpec((bmh, 2 * d_in), lambda i: (i, 0)),
            pl.BlockSpec((2 * d_in, 2 * d_hid), lambda i: (0, 0)),
            pl.BlockSpec((2 * d_hid, 2 * d_out), lambda i: (0, 0)),
            pl.BlockSpec((2 * d_hid, 2 * d_out), lambda i: (0, 0)),
            pl.BlockSpec((1, 2 * d_out), lambda i: (0, 0)),
        ],
        out_specs=[
            pl.BlockSpec((bmh, 2 * d_out), lambda i: (i, 0)),
            pl.BlockSpec((bmh, 2 * d_out), lambda i: (i, 0)),
        ],
        out_shape=[jax.ShapeDtypeStruct((n // 2, 2 * d_out), jnp.float32)] * 2,
    )(hs, *p1s, dinv1, wn1p, _pair(W_self2), _pair(W_neigh2),
      jnp.concatenate([b2, b2]).reshape(1, 2 * d_out))

    t2 = jnp.reshape(t2p, (n, d_out))
    seg2 = _seg_sum_call(n_pad, d_out, iters, 1)
    p2 = seg2(src2, dst2, t2)
    if isinstance(p2, (list, tuple)):
        p2 = p2[0]
    p2 = jnp.reshape(p2, (NC, n_pad // 2, 2 * d_out))
    dinv2 = jnp.reshape(jnp.broadcast_to(dinv[:, None], (n, d_out)),
                        (n // 2, 2 * d_out))

    outp = pl.pallas_call(
        _tc2_body,
        grid=grid,
        in_specs=[
            pl.BlockSpec((bmh, 2 * d_out), lambda i: (i, 0)),
            pl.BlockSpec((NC, bmh, 2 * d_out), lambda i: (0, i, 0)),
            pl.BlockSpec((bmh, 2 * d_out), lambda i: (i, 0)),
        ],
        out_specs=pl.BlockSpec((bmh, 2 * d_out), lambda i: (i, 0)),
        out_shape=jax.ShapeDtypeStruct((n // 2, 2 * d_out), jnp.float32),
    )(s2p, p2, dinv2)
    return jnp.reshape(outp, (n, d_out))
